# baseline (speedup 1.0000x reference)
import functools

import jax
import jax.numpy as jnp
from jax import lax
from jax.experimental import pallas as pl
from jax.experimental.pallas import tpu as pltpu
from jax.experimental.pallas import tpu_sc as plsc

N = 10000
E = 320000
D = 128
DOUT = 64
NC, NS = 2, 16
NW = NC * NS
K = 80
EPW = E // NW
STEPS = EPW // K
NBUF = 3
ROWS_PT = 624
TAIL = N - NS * ROWS_PT
NPAD = 10240
DPT = NPAD // NS

def _deg_body(dst_hbm, deg_hbm, dacc, dst_v, ones_v, zero_v):
    c = lax.axis_index("c")
    s = lax.axis_index("s")
    wid = c * NS + s
    for k in range(K // 16):
        ones_v[pl.ds(k * 16, 16)] = jnp.ones((16,), jnp.float32)
    for k in range(DPT // 16):
        zero_v[pl.ds(k * 16, 16)] = jnp.zeros((16,), jnp.float32)
    pltpu.sync_copy(zero_v, dacc.at[pl.ds(s * DPT, DPT)])
    plsc.subcore_barrier()
    pltpu.sync_copy(dst_hbm.at[wid], dst_v)

    def body(j, _):
        pltpu.sync_copy(ones_v, dacc.at[dst_v.at[j]], add=True)
        return 0

    lax.fori_loop(0, STEPS, body, 0)
    plsc.subcore_barrier()
    pltpu.sync_copy(dacc.at[pl.ds(s * DPT, DPT)],
                    deg_hbm.at[c, pl.ds(s * DPT, DPT)])


def _prop_body(g_hbm, packed_hbm, zeros_hbm, out_hbm,
               acc, packed_v, schunks, dchunks, rows_v, gs, ss):
    c = lax.axis_index("c")
    s = lax.axis_index("s")
    wid = c * NS + s
    @pl.when(c == 0)
    def _():
        pltpu.sync_copy(g_hbm.at[pl.ds(s * ROWS_PT, ROWS_PT)],
                        acc.at[pl.ds(s * ROWS_PT, ROWS_PT)])

    @pl.when(c != 0)
    def _():
        pltpu.sync_copy(zeros_hbm.at[pl.ds(s * ROWS_PT, ROWS_PT)],
                        acc.at[pl.ds(s * ROWS_PT, ROWS_PT)])

    @pl.when((c == 0) & (s == 0))
    def _():
        pltpu.sync_copy(g_hbm.at[pl.ds(NS * ROWS_PT, TAIL)],
                        acc.at[pl.ds(NS * ROWS_PT, TAIL)])

    @pl.when((c != 0) & (s == 0))
    def _():
        pltpu.sync_copy(zeros_hbm.at[pl.ds(NS * ROWS_PT, TAIL)],
                        acc.at[pl.ds(NS * ROWS_PT, TAIL)])

    pltpu.sync_copy(packed_hbm.at[pl.ds(wid * EPW, EPW)], packed_v)
    plsc.subcore_barrier()

    def fill_s(j, b):
        for k in range(K // 16):
            v = packed_v[pl.ds(j * K + k * 16, 16)]
            schunks[b][pl.ds(k * 16, 16)] = v & 0xFFFF

    def fill_d(j, b):
        for k in range(K // 16):
            v = packed_v[pl.ds(j * K + k * 16, 16)]
            dchunks[b][pl.ds(k * 16, 16)] = lax.shift_right_logical(v, 16)

    def start_gather(j, b):
        fill_s(j, b)
        pltpu.async_copy(g_hbm.at[schunks[b]], rows_v.at[b], gs[b])

    def wait_gather(b):
        pltpu.make_async_copy(g_hbm.at[schunks[b]], rows_v.at[b],
                              gs[b]).wait()

    def start_scatter(j, b):
        fill_d(j, b)
        pltpu.async_copy(rows_v.at[b], acc.at[dchunks[b]], ss[b], add=True)

    def wait_scatter(b):
        pltpu.make_async_copy(rows_v.at[b], acc.at[dchunks[b]], ss[b]).wait()

    start_gather(0, 0)
    start_gather(1, 1)
    wait_gather(0)
    start_scatter(0, 0)
    start_gather(2, 2)

    def outer(o, _):
        for t in range(NBUF):
            j = NBUF * o + 1 + t
            b = (1 + t) % NBUF
            bp = t % NBUF
            wait_gather(b)
            start_scatter(j, b)
            wait_scatter(bp)
            start_gather(j + 2, bp)
        return 0

    MAINI = (STEPS - 5) // NBUF
    lax.fori_loop(0, MAINI, outer, 0)
    for j in range(NBUF * MAINI + 1, STEPS):
        b = j % NBUF
        bp = (j - 1) % NBUF
        wait_gather(b)
        start_scatter(j, b)
        wait_scatter(bp)
        if j + 2 < STEPS:
            start_gather(j + 2, bp)
    wait_scatter((STEPS - 1) % NBUF)
    plsc.subcore_barrier()
    pltpu.sync_copy(acc.at[pl.ds(s * ROWS_PT, ROWS_PT)],
                    out_hbm.at[c, pl.ds(s * ROWS_PT, ROWS_PT)])

    @pl.when(s == 0)
    def _():
        pltpu.sync_copy(acc.at[pl.ds(NS * ROWS_PT, TAIL)],
                        out_hbm.at[c, pl.ds(NS * ROWS_PT, TAIL)])


@functools.lru_cache(maxsize=None)
def _sc_kernels():
    mesh = plsc.VectorSubcoreMesh(core_axis_name="c", subcore_axis_name="s")
    deg_k = pl.kernel(
        _deg_body,
        out_type=jax.ShapeDtypeStruct((NC, NPAD), jnp.float32),
        mesh=mesh,
        scratch_types=[
            pltpu.VMEM_SHARED((NPAD,), jnp.float32),
            pltpu.VMEM((STEPS, K), jnp.int32),
            pltpu.VMEM((K,), jnp.float32),
            pltpu.VMEM((DPT,), jnp.float32),
        ],
    )
    prop_k = pl.kernel(
        _prop_body,
        out_type=jax.ShapeDtypeStruct((NC, N, D), jnp.float32),
        mesh=mesh,
        scratch_types=[
            pltpu.VMEM_SHARED((N, D), jnp.float32),
            pltpu.VMEM((EPW,), jnp.int32),
            [pltpu.VMEM((K,), jnp.int32)] * NBUF,
            [pltpu.VMEM((K,), jnp.int32)] * NBUF,
            pltpu.VMEM((NBUF, K, D), jnp.float32),
            [pltpu.SemaphoreType.DMA] * NBUF,
            [pltpu.SemaphoreType.DMA] * NBUF,
        ],
    )
    return deg_k, prop_k


_BS = 2000


def _tc_prep_body(deg_ref, x_ref, w_ref, g_ref):
    d = deg_ref[...]
    dinv = lax.rsqrt(d[:, 0:1] + d[:, 1:2] + 1.0)
    g_ref[...] = dinv * jnp.dot(x_ref[...], w_ref[...],
                                preferred_element_type=jnp.float32)


def _tc_layer_body(deg_ref, s_ref, b_ref, w_ref, g_ref):
    d = deg_ref[...]
    dinv = lax.rsqrt(d[:, 0:1] + d[:, 1:2] + 1.0)
    h = jax.nn.relu(dinv * (s_ref[0] + s_ref[1]) + b_ref[...])
    g_ref[...] = dinv * jnp.dot(h, w_ref[...],
                                preferred_element_type=jnp.float32)


def _tc_final_body(deg_ref, s_ref, b_ref, wfc_ref, bfc_ref, out_ref, acc):
    i = pl.program_id(0)

    @pl.when(i == 0)
    def _():
        acc[...] = jnp.zeros_like(acc)

    d = deg_ref[...]
    dinv = lax.rsqrt(d[:, 0:1] + d[:, 1:2] + 1.0)
    h = jax.nn.relu(dinv * (s_ref[0] + s_ref[1]) + b_ref[...])
    acc[...] += jnp.sum(h, axis=0, keepdims=True)

    @pl.when(i == pl.num_programs(0) - 1)
    def _():
        pooled = acc[...] * (1.0 / N)
        out_ref[...] = jnp.dot(pooled, wfc_ref[...],
                               preferred_element_type=jnp.float32) + bfc_ref[...]


def _tc_prep(deg2, x, w):
    grid = N // _BS
    return pl.pallas_call(
        _tc_prep_body,
        grid=(grid,),
        in_specs=[
            pl.BlockSpec((_BS, 2), lambda i: (i, 0)),
            pl.BlockSpec((_BS, D), lambda i: (i, 0)),
            pl.BlockSpec((D, D), lambda i: (0, 0)),
        ],
        out_specs=pl.BlockSpec((_BS, D), lambda i: (i, 0)),
        out_shape=jax.ShapeDtypeStruct((N, D), jnp.float32),
    )(deg2, x, w)


def _tc_layer(deg2, s, b, w):
    grid = N // _BS
    return pl.pallas_call(
        _tc_layer_body,
        grid=(grid,),
        in_specs=[
            pl.BlockSpec((_BS, 2), lambda i: (i, 0)),
            pl.BlockSpec((NC, _BS, D), lambda i: (0, i, 0)),
            pl.BlockSpec((1, D), lambda i: (0, 0)),
            pl.BlockSpec((D, D), lambda i: (0, 0)),
        ],
        out_specs=pl.BlockSpec((_BS, D), lambda i: (i, 0)),
        out_shape=jax.ShapeDtypeStruct((N, D), jnp.float32),
    )(deg2, s, b, w)


def _tc_final(deg2, s, b, wfc, bfc):
    grid = N // _BS
    return pl.pallas_call(
        _tc_final_body,
        grid=(grid,),
        in_specs=[
            pl.BlockSpec((_BS, 2), lambda i: (i, 0)),
            pl.BlockSpec((NC, _BS, D), lambda i: (0, i, 0)),
            pl.BlockSpec((1, D), lambda i: (0, 0)),
            pl.BlockSpec((D, DOUT), lambda i: (0, 0)),
            pl.BlockSpec((1, DOUT), lambda i: (0, 0)),
        ],
        out_specs=pl.BlockSpec((1, DOUT), lambda i: (0, 0)),
        out_shape=jax.ShapeDtypeStruct((1, DOUT), jnp.float32),
        scratch_shapes=[pltpu.VMEM((1, D), jnp.float32)],
    )(deg2, s, b, wfc, bfc)


def kernel(x, edge_index, W1, b1, W2, b2, W3, b3, Wfc, bfc):
    src = edge_index[0]
    dst = edge_index[1]
    dst3 = dst.reshape(NW, STEPS, K)
    packed = jnp.bitwise_or(jnp.left_shift(dst, 16), src)
    zeros = jnp.zeros((N, D), jnp.float32)

    deg_kernel, prop_kernel = _sc_kernels()
    degp = deg_kernel(dst3)
    deg2 = jnp.transpose(degp)[:N]

    g = _tc_prep(deg2, x, W1)
    s = prop_kernel(g, packed, zeros)
    g = _tc_layer(deg2, s, b1.reshape(1, D), W2)
    s = prop_kernel(g, packed, zeros)
    g = _tc_layer(deg2, s, b2.reshape(1, D), W3)
    s = prop_kernel(g, packed, zeros)
    return _tc_final(deg2, s, b3.reshape(1, D), Wfc, bfc.reshape(1, DOUT))


# speedup vs baseline: 1.0114x; 1.0114x over previous
import functools

import jax
import jax.numpy as jnp
from jax import lax
from jax.experimental import pallas as pl
from jax.experimental.pallas import tpu as pltpu
from jax.experimental.pallas import tpu_sc as plsc

N = 10000
E = 320000
D = 128
DOUT = 64
NC, NS = 2, 16
NW = NC * NS
K = 80
EPW = E // NW
STEPS = EPW // K
NBUF = 3
ROWS_PT = 624
TAIL = N - NS * ROWS_PT
ZR = 16
NPAD = 10240
DPT = NPAD // NS

def _deg_body(dst_hbm, deg_hbm, dacc, dst_v, ones_v, zero_v):
    c = lax.axis_index("c")
    s = lax.axis_index("s")
    wid = c * NS + s
    for k in range(K // 16):
        ones_v[pl.ds(k * 16, 16)] = jnp.ones((16,), jnp.float32)
    for k in range(DPT // 16):
        zero_v[pl.ds(k * 16, 16)] = jnp.zeros((16,), jnp.float32)
    pltpu.sync_copy(zero_v, dacc.at[pl.ds(s * DPT, DPT)])
    plsc.subcore_barrier()
    pltpu.sync_copy(dst_hbm.at[wid], dst_v)

    def body(j, _):
        pltpu.sync_copy(ones_v, dacc.at[dst_v.at[j]], add=True)
        return 0

    lax.fori_loop(0, STEPS, body, 0)
    plsc.subcore_barrier()
    pltpu.sync_copy(dacc.at[pl.ds(s * DPT, DPT)],
                    deg_hbm.at[c, pl.ds(s * DPT, DPT)])


def _prop_body(g_hbm, packed_hbm, out_hbm,
               acc, packed_v, schunks, dchunks, zbuf, rows_v, gs, ss):
    c = lax.axis_index("c")
    s = lax.axis_index("s")
    wid = c * NS + s
    for r in range(ZR):
        for cc in range(D // 16):
            zbuf[r, pl.ds(cc * 16, 16)] = jnp.zeros((16,), jnp.float32)
    for r in range(ROWS_PT // ZR):
        pltpu.sync_copy(zbuf, acc.at[pl.ds(s * ROWS_PT + r * ZR, ZR)])

    @pl.when(s == 0)
    def _():
        pltpu.sync_copy(zbuf, acc.at[pl.ds(NS * ROWS_PT, TAIL)])

    pltpu.sync_copy(packed_hbm.at[pl.ds(wid * EPW, EPW)], packed_v)
    plsc.subcore_barrier()

    def fill_s(j, b):
        for k in range(K // 16):
            v = packed_v[pl.ds(j * K + k * 16, 16)]
            schunks[b][pl.ds(k * 16, 16)] = v & 0xFFFF

    def fill_d(j, b):
        for k in range(K // 16):
            v = packed_v[pl.ds(j * K + k * 16, 16)]
            dchunks[b][pl.ds(k * 16, 16)] = lax.shift_right_logical(v, 16)

    def start_gather(j, b):
        fill_s(j, b)
        pltpu.async_copy(g_hbm.at[schunks[b]], rows_v.at[b], gs[b])

    def wait_gather(b):
        pltpu.make_async_copy(g_hbm.at[schunks[b]], rows_v.at[b],
                              gs[b]).wait()

    def start_scatter(j, b):
        fill_d(j, b)
        pltpu.async_copy(rows_v.at[b], acc.at[dchunks[b]], ss[b], add=True)

    def wait_scatter(b):
        pltpu.make_async_copy(rows_v.at[b], acc.at[dchunks[b]], ss[b]).wait()

    start_gather(0, 0)
    start_gather(1, 1)
    wait_gather(0)
    start_scatter(0, 0)
    start_gather(2, 2)

    def outer(o, _):
        for t in range(NBUF):
            j = NBUF * o + 1 + t
            b = (1 + t) % NBUF
            bp = t % NBUF
            wait_gather(b)
            start_scatter(j, b)
            wait_scatter(bp)
            start_gather(j + 2, bp)
        return 0

    MAINI = (STEPS - 5) // NBUF
    lax.fori_loop(0, MAINI, outer, 0)
    for j in range(NBUF * MAINI + 1, STEPS):
        b = j % NBUF
        bp = (j - 1) % NBUF
        wait_gather(b)
        start_scatter(j, b)
        wait_scatter(bp)
        if j + 2 < STEPS:
            start_gather(j + 2, bp)
    wait_scatter((STEPS - 1) % NBUF)
    plsc.subcore_barrier()
    pltpu.sync_copy(acc.at[pl.ds(s * ROWS_PT, ROWS_PT)],
                    out_hbm.at[c, pl.ds(s * ROWS_PT, ROWS_PT)])

    @pl.when(s == 0)
    def _():
        pltpu.sync_copy(acc.at[pl.ds(NS * ROWS_PT, TAIL)],
                        out_hbm.at[c, pl.ds(NS * ROWS_PT, TAIL)])


@functools.lru_cache(maxsize=None)
def _sc_kernels():
    mesh = plsc.VectorSubcoreMesh(core_axis_name="c", subcore_axis_name="s")
    deg_k = pl.kernel(
        _deg_body,
        out_type=jax.ShapeDtypeStruct((NC, NPAD), jnp.float32),
        mesh=mesh,
        scratch_types=[
            pltpu.VMEM_SHARED((NPAD,), jnp.float32),
            pltpu.VMEM((STEPS, K), jnp.int32),
            pltpu.VMEM((K,), jnp.float32),
            pltpu.VMEM((DPT,), jnp.float32),
        ],
    )
    prop_k = pl.kernel(
        _prop_body,
        out_type=jax.ShapeDtypeStruct((NC, N, D), jnp.float32),
        mesh=mesh,
        scratch_types=[
            pltpu.VMEM_SHARED((N, D), jnp.float32),
            pltpu.VMEM((EPW,), jnp.int32),
            [pltpu.VMEM((K,), jnp.int32)] * NBUF,
            [pltpu.VMEM((K,), jnp.int32)] * NBUF,
            pltpu.VMEM((ZR, D), jnp.float32),
            pltpu.VMEM((NBUF, K, D), jnp.float32),
            [pltpu.SemaphoreType.DMA] * NBUF,
            [pltpu.SemaphoreType.DMA] * NBUF,
        ],
    )
    return deg_k, prop_k


_BS = 2000


def _tc_prep_body(deg_ref, x_ref, w_ref, g_ref):
    d = deg_ref[...]
    dinv = lax.rsqrt(d[:, 0:1] + d[:, 1:2] + 1.0)
    g_ref[...] = dinv * jnp.dot(x_ref[...], w_ref[...],
                                preferred_element_type=jnp.float32)


def _tc_layer_body(deg_ref, s_ref, gin_ref, b_ref, w_ref, g_ref):
    d = deg_ref[...]
    dinv = lax.rsqrt(d[:, 0:1] + d[:, 1:2] + 1.0)
    h = jax.nn.relu(dinv * (s_ref[0] + s_ref[1] + gin_ref[...]) + b_ref[...])
    g_ref[...] = dinv * jnp.dot(h, w_ref[...],
                                preferred_element_type=jnp.float32)


def _tc_final_body(deg_ref, s_ref, gin_ref, b_ref, wfc_ref, bfc_ref,
                   out_ref, acc):
    i = pl.program_id(0)

    @pl.when(i == 0)
    def _():
        acc[...] = jnp.zeros_like(acc)

    d = deg_ref[...]
    dinv = lax.rsqrt(d[:, 0:1] + d[:, 1:2] + 1.0)
    h = jax.nn.relu(dinv * (s_ref[0] + s_ref[1] + gin_ref[...]) + b_ref[...])
    acc[...] += jnp.sum(h, axis=0, keepdims=True)

    @pl.when(i == pl.num_programs(0) - 1)
    def _():
        pooled = acc[...] * (1.0 / N)
        out_ref[...] = jnp.dot(pooled, wfc_ref[...],
                               preferred_element_type=jnp.float32) + bfc_ref[...]


def _tc_prep(deg2, x, w):
    grid = N // _BS
    return pl.pallas_call(
        _tc_prep_body,
        grid=(grid,),
        in_specs=[
            pl.BlockSpec((_BS, 2), lambda i: (i, 0)),
            pl.BlockSpec((_BS, D), lambda i: (i, 0)),
            pl.BlockSpec((D, D), lambda i: (0, 0)),
        ],
        out_specs=pl.BlockSpec((_BS, D), lambda i: (i, 0)),
        out_shape=jax.ShapeDtypeStruct((N, D), jnp.float32),
    )(deg2, x, w)


def _tc_layer(deg2, s, gin, b, w):
    grid = N // _BS
    return pl.pallas_call(
        _tc_layer_body,
        grid=(grid,),
        in_specs=[
            pl.BlockSpec((_BS, 2), lambda i: (i, 0)),
            pl.BlockSpec((NC, _BS, D), lambda i: (0, i, 0)),
            pl.BlockSpec((_BS, D), lambda i: (i, 0)),
            pl.BlockSpec((1, D), lambda i: (0, 0)),
            pl.BlockSpec((D, D), lambda i: (0, 0)),
        ],
        out_specs=pl.BlockSpec((_BS, D), lambda i: (i, 0)),
        out_shape=jax.ShapeDtypeStruct((N, D), jnp.float32),
    )(deg2, s, gin, b, w)


def _tc_final(deg2, s, gin, b, wfc, bfc):
    grid = N // _BS
    return pl.pallas_call(
        _tc_final_body,
        grid=(grid,),
        in_specs=[
            pl.BlockSpec((_BS, 2), lambda i: (i, 0)),
            pl.BlockSpec((NC, _BS, D), lambda i: (0, i, 0)),
            pl.BlockSpec((_BS, D), lambda i: (i, 0)),
            pl.BlockSpec((1, D), lambda i: (0, 0)),
            pl.BlockSpec((D, DOUT), lambda i: (0, 0)),
            pl.BlockSpec((1, DOUT), lambda i: (0, 0)),
        ],
        out_specs=pl.BlockSpec((1, DOUT), lambda i: (0, 0)),
        out_shape=jax.ShapeDtypeStruct((1, DOUT), jnp.float32),
        scratch_shapes=[pltpu.VMEM((1, D), jnp.float32)],
    )(deg2, s, gin, b, wfc, bfc)


def kernel(x, edge_index, W1, b1, W2, b2, W3, b3, Wfc, bfc):
    src = edge_index[0]
    dst = edge_index[1]
    dst3 = dst.reshape(NW, STEPS, K)
    packed = jnp.bitwise_or(jnp.left_shift(dst, 16), src)

    deg_kernel, prop_kernel = _sc_kernels()
    degp = deg_kernel(dst3)
    deg2 = jnp.transpose(degp)[:N]

    g = _tc_prep(deg2, x, W1)
    s = prop_kernel(g, packed)
    g = _tc_layer(deg2, s, g, b1.reshape(1, D), W2)
    s = prop_kernel(g, packed)
    g = _tc_layer(deg2, s, g, b2.reshape(1, D), W3)
    s = prop_kernel(g, packed)
    return _tc_final(deg2, s, g, b3.reshape(1, D), Wfc, bfc.reshape(1, DOUT))


# speedup vs baseline: 1.0278x; 1.0162x over previous
import functools

import jax
import jax.numpy as jnp
from jax import lax
from jax.experimental import pallas as pl
from jax.experimental.pallas import tpu as pltpu
from jax.experimental.pallas import tpu_sc as plsc

N = 10000
E = 320000
D = 128
DOUT = 64
NC, NS = 2, 16
NW = NC * NS
K = 80
EPW = E // NW
STEPS = EPW // K
NBUF = 3
ROWS_PT = 624
TAIL = N - NS * ROWS_PT
ZR = 16
NPAD = 10240
DPT = NPAD // NS

def _deg_body(dst_hbm, deg_hbm, dacc, dst_v, ones_v, zero_v, sem):
    c = lax.axis_index("c")
    s = lax.axis_index("s")
    wid = c * NS + s
    for k in range(K // 16):
        ones_v[pl.ds(k * 16, 16)] = jnp.ones((16,), jnp.float32)
    for k in range(DPT // 16):
        zero_v[pl.ds(k * 16, 16)] = jnp.zeros((16,), jnp.float32)
    pltpu.sync_copy(zero_v, dacc.at[pl.ds(s * DPT, DPT)])
    plsc.subcore_barrier()
    pltpu.sync_copy(dst_hbm.at[wid], dst_v)

    def fire(j, _):
        pltpu.async_copy(ones_v, dacc.at[dst_v.at[j]], sem, add=True)
        return 0

    lax.fori_loop(0, STEPS, fire, 0)

    def drain(j, _):
        pltpu.make_async_copy(ones_v, dacc.at[dst_v.at[j]], sem).wait()
        return 0

    lax.fori_loop(0, STEPS, drain, 0)
    plsc.subcore_barrier()
    pltpu.sync_copy(dacc.at[pl.ds(s * DPT, DPT)],
                    deg_hbm.at[c, pl.ds(s * DPT, DPT)])


def _prop_body(g_hbm, packed_hbm, out_hbm,
               acc, packed_v, schunks, dchunks, zbuf, rows_v, gs, ss):
    c = lax.axis_index("c")
    s = lax.axis_index("s")
    wid = c * NS + s
    for r in range(ZR):
        for cc in range(D // 16):
            zbuf[r, pl.ds(cc * 16, 16)] = jnp.zeros((16,), jnp.float32)
    for r in range(ROWS_PT // ZR):
        pltpu.sync_copy(zbuf, acc.at[pl.ds(s * ROWS_PT + r * ZR, ZR)])

    @pl.when(s == 0)
    def _():
        pltpu.sync_copy(zbuf, acc.at[pl.ds(NS * ROWS_PT, TAIL)])

    pltpu.sync_copy(packed_hbm.at[pl.ds(wid * EPW, EPW)], packed_v)
    plsc.subcore_barrier()

    def fill_s(j, b):
        for k in range(K // 16):
            v = packed_v[pl.ds(j * K + k * 16, 16)]
            schunks[b][pl.ds(k * 16, 16)] = v & 0xFFFF

    def fill_d(j, b):
        for k in range(K // 16):
            v = packed_v[pl.ds(j * K + k * 16, 16)]
            dchunks[b][pl.ds(k * 16, 16)] = lax.shift_right_logical(v, 16)

    def start_gather(j, b):
        fill_s(j, b)
        pltpu.async_copy(g_hbm.at[schunks[b]], rows_v.at[b], gs[b])

    def wait_gather(b):
        pltpu.make_async_copy(g_hbm.at[schunks[b]], rows_v.at[b],
                              gs[b]).wait()

    def start_scatter(j, b):
        fill_d(j, b)
        pltpu.async_copy(rows_v.at[b], acc.at[dchunks[b]], ss[b], add=True)

    def wait_scatter(b):
        pltpu.make_async_copy(rows_v.at[b], acc.at[dchunks[b]], ss[b]).wait()

    start_gather(0, 0)
    start_gather(1, 1)
    wait_gather(0)
    start_scatter(0, 0)
    start_gather(2, 2)

    def outer(o, _):
        for t in range(NBUF):
            j = NBUF * o + 1 + t
            b = (1 + t) % NBUF
            bp = t % NBUF
            wait_gather(b)
            start_scatter(j, b)
            wait_scatter(bp)
            start_gather(j + 2, bp)
        return 0

    MAINI = (STEPS - 5) // NBUF
    lax.fori_loop(0, MAINI, outer, 0)
    for j in range(NBUF * MAINI + 1, STEPS):
        b = j % NBUF
        bp = (j - 1) % NBUF
        wait_gather(b)
        start_scatter(j, b)
        wait_scatter(bp)
        if j + 2 < STEPS:
            start_gather(j + 2, bp)
    wait_scatter((STEPS - 1) % NBUF)
    plsc.subcore_barrier()
    pltpu.sync_copy(acc.at[pl.ds(s * ROWS_PT, ROWS_PT)],
                    out_hbm.at[c, pl.ds(s * ROWS_PT, ROWS_PT)])

    @pl.when(s == 0)
    def _():
        pltpu.sync_copy(acc.at[pl.ds(NS * ROWS_PT, TAIL)],
                        out_hbm.at[c, pl.ds(NS * ROWS_PT, TAIL)])


@functools.lru_cache(maxsize=None)
def _sc_kernels():
    mesh = plsc.VectorSubcoreMesh(core_axis_name="c", subcore_axis_name="s")
    deg_k = pl.kernel(
        _deg_body,
        out_type=jax.ShapeDtypeStruct((NC, NPAD), jnp.float32),
        mesh=mesh,
        scratch_types=[
            pltpu.VMEM_SHARED((NPAD,), jnp.float32),
            pltpu.VMEM((STEPS, K), jnp.int32),
            pltpu.VMEM((K,), jnp.float32),
            pltpu.VMEM((DPT,), jnp.float32),
            pltpu.SemaphoreType.DMA,
        ],
    )
    prop_k = pl.kernel(
        _prop_body,
        out_type=jax.ShapeDtypeStruct((NC, N, D), jnp.float32),
        mesh=mesh,
        scratch_types=[
            pltpu.VMEM_SHARED((N, D), jnp.float32),
            pltpu.VMEM((EPW,), jnp.int32),
            [pltpu.VMEM((K,), jnp.int32)] * NBUF,
            [pltpu.VMEM((K,), jnp.int32)] * NBUF,
            pltpu.VMEM((ZR, D), jnp.float32),
            pltpu.VMEM((NBUF, K, D), jnp.float32),
            [pltpu.SemaphoreType.DMA] * NBUF,
            [pltpu.SemaphoreType.DMA] * NBUF,
        ],
    )
    return deg_k, prop_k


_BS = 2000


def _tc_mm_body(x_ref, w_ref, u_ref):
    u_ref[...] = jnp.dot(x_ref[...], w_ref[...],
                         preferred_element_type=jnp.float32)


def _tc_scale_body(deg_ref, u_ref, g_ref):
    d = deg_ref[...]
    dinv = lax.rsqrt(d[:, 0:1] + d[:, 1:2] + 1.0)
    g_ref[...] = dinv * u_ref[...]


def _tc_layer_body(deg_ref, s_ref, gin_ref, b_ref, w_ref, g_ref):
    d = deg_ref[...]
    dinv = lax.rsqrt(d[:, 0:1] + d[:, 1:2] + 1.0)
    h = jax.nn.relu(dinv * (s_ref[0] + s_ref[1] + gin_ref[...]) + b_ref[...])
    g_ref[...] = dinv * jnp.dot(h, w_ref[...],
                                preferred_element_type=jnp.float32)


def _tc_final_body(deg_ref, s_ref, gin_ref, b_ref, wfc_ref, bfc_ref,
                   out_ref, acc):
    i = pl.program_id(0)

    @pl.when(i == 0)
    def _():
        acc[...] = jnp.zeros_like(acc)

    d = deg_ref[...]
    dinv = lax.rsqrt(d[:, 0:1] + d[:, 1:2] + 1.0)
    h = jax.nn.relu(dinv * (s_ref[0] + s_ref[1] + gin_ref[...]) + b_ref[...])
    acc[...] += jnp.sum(h, axis=0, keepdims=True)

    @pl.when(i == pl.num_programs(0) - 1)
    def _():
        pooled = acc[...] * (1.0 / N)
        out_ref[...] = jnp.dot(pooled, wfc_ref[...],
                               preferred_element_type=jnp.float32) + bfc_ref[...]


def _tc_mm(x, w):
    grid = N // _BS
    return pl.pallas_call(
        _tc_mm_body,
        grid=(grid,),
        in_specs=[
            pl.BlockSpec((_BS, D), lambda i: (i, 0)),
            pl.BlockSpec((D, D), lambda i: (0, 0)),
        ],
        out_specs=pl.BlockSpec((_BS, D), lambda i: (i, 0)),
        out_shape=jax.ShapeDtypeStruct((N, D), jnp.float32),
    )(x, w)


def _tc_scale(deg2, u):
    grid = N // _BS
    return pl.pallas_call(
        _tc_scale_body,
        grid=(grid,),
        in_specs=[
            pl.BlockSpec((_BS, 2), lambda i: (i, 0)),
            pl.BlockSpec((_BS, D), lambda i: (i, 0)),
        ],
        out_specs=pl.BlockSpec((_BS, D), lambda i: (i, 0)),
        out_shape=jax.ShapeDtypeStruct((N, D), jnp.float32),
    )(deg2, u)


def _tc_layer(deg2, s, gin, b, w):
    grid = N // _BS
    return pl.pallas_call(
        _tc_layer_body,
        grid=(grid,),
        in_specs=[
            pl.BlockSpec((_BS, 2), lambda i: (i, 0)),
            pl.BlockSpec((NC, _BS, D), lambda i: (0, i, 0)),
            pl.BlockSpec((_BS, D), lambda i: (i, 0)),
            pl.BlockSpec((1, D), lambda i: (0, 0)),
            pl.BlockSpec((D, D), lambda i: (0, 0)),
        ],
        out_specs=pl.BlockSpec((_BS, D), lambda i: (i, 0)),
        out_shape=jax.ShapeDtypeStruct((N, D), jnp.float32),
    )(deg2, s, gin, b, w)


def _tc_final(deg2, s, gin, b, wfc, bfc):
    grid = N // _BS
    return pl.pallas_call(
        _tc_final_body,
        grid=(grid,),
        in_specs=[
            pl.BlockSpec((_BS, 2), lambda i: (i, 0)),
            pl.BlockSpec((NC, _BS, D), lambda i: (0, i, 0)),
            pl.BlockSpec((_BS, D), lambda i: (i, 0)),
            pl.BlockSpec((1, D), lambda i: (0, 0)),
            pl.BlockSpec((D, DOUT), lambda i: (0, 0)),
            pl.BlockSpec((1, DOUT), lambda i: (0, 0)),
        ],
        out_specs=pl.BlockSpec((1, DOUT), lambda i: (0, 0)),
        out_shape=jax.ShapeDtypeStruct((1, DOUT), jnp.float32),
        scratch_shapes=[pltpu.VMEM((1, D), jnp.float32)],
    )(deg2, s, gin, b, wfc, bfc)


def kernel(x, edge_index, W1, b1, W2, b2, W3, b3, Wfc, bfc):
    src = edge_index[0]
    dst = edge_index[1]
    dst3 = dst.reshape(NW, STEPS, K)
    packed = jnp.bitwise_or(jnp.left_shift(dst, 16), src)

    deg_kernel, prop_kernel = _sc_kernels()
    degp = deg_kernel(dst3)
    u = _tc_mm(x, W1)
    deg2 = jnp.transpose(degp)[:N]

    g = _tc_scale(deg2, u)
    s = prop_kernel(g, packed)
    g = _tc_layer(deg2, s, g, b1.reshape(1, D), W2)
    s = prop_kernel(g, packed)
    g = _tc_layer(deg2, s, g, b2.reshape(1, D), W3)
    s = prop_kernel(g, packed)
    return _tc_final(deg2, s, g, b3.reshape(1, D), Wfc, bfc.reshape(1, DOUT))
